# baseline (speedup 1.0000x reference)
import jax
import jax.numpy as jnp
import numpy as np
from jax.experimental import pallas as pl

_ROWS, _COLS = 128, 100000
_RB = 8
_NBLK = _ROWS // _RB
_NT_FULL = _COLS // 128
_PARTIAL = _COLS - _NT_FULL * 128
_NT_PAD = _NT_FULL + 1
_NC4 = _NT_FULL // 4
_CW = 1024
_NWCH = _COLS // _CW
_WTAIL = _COLS - _NWCH * _CW
_BIG = np.int32(2 ** 30)
_NEG = np.float32(-np.inf)


_NOISE_CACHE = []


def _gumbel_tiles():
    if not _NOISE_CACHE:
        u = jax.random.uniform(jax.random.key(42), (_ROWS, _COLS),
                               dtype=jnp.float32)
        g = -jnp.log(-jnp.log(u + 1e-10) + 1e-10)
        gp = jnp.pad(g, ((0, 0), (0, _NT_PAD * 128 - _COLS)))
        gt = gp.reshape(_NBLK, _RB, _NT_PAD, 128).transpose(0, 2, 1, 3)
        _NOISE_CACHE.append(gt)
    return _NOISE_CACHE[0]


def _body(x_ref, g_ref, out_ref):
    l_io = jax.lax.broadcasted_iota(jnp.int32, (_RB, 128), 1)

    def step(j, carry):
        accs = list(carry)
        t0 = j * 4
        for k in range(4):
            t = t0 + k
            z = x_ref[:, pl.ds(t * 128, 128)] + g_ref[0, t]
            m_k, c_k = accs[2 * k], accs[2 * k + 1]
            upd = z > m_k
            accs[2 * k] = jnp.where(upd, z, m_k)
            accs[2 * k + 1] = jnp.where(upd, t, c_k)
        return tuple(accs)

    init = []
    for _ in range(4):
        init.append(jnp.full((_RB, 128), _NEG, jnp.float32))
        init.append(jnp.zeros((_RB, 128), jnp.int32))
    accs = jax.lax.fori_loop(0, _NC4, step, tuple(init))

    z = x_ref[:, pl.ds(_NC4 * 4 * 128, 128)] + g_ref[0, _NC4 * 4]
    m_0, c_0 = accs[0], accs[1]
    upd = z > m_0
    m_list = [jnp.where(upd, z, m_0)] + [accs[2], accs[4], accs[6]]
    c_list = [jnp.where(upd, _NC4 * 4, c_0)] + [accs[3], accs[5], accs[7]]

    M, C = m_list[0], c_list[0]
    for k in range(1, 4):
        better = (m_list[k] > M) | ((m_list[k] == M) & (c_list[k] < C))
        M = jnp.where(better, m_list[k], M)
        C = jnp.where(better, c_list[k], C)

    col_full = C * 128 + l_io
    m_main = jnp.max(M, axis=1, keepdims=True)

    l_io_p = jax.lax.broadcasted_iota(jnp.int32, (_RB, _PARTIAL), 1)
    z_p = (x_ref[:, pl.ds(_NT_FULL * 128, _PARTIAL)]
           + g_ref[0, _NT_FULL, :, : _PARTIAL])
    m_part = jnp.max(z_p, axis=1, keepdims=True)

    m = jnp.maximum(m_main, m_part)
    cand_main = jnp.min(jnp.where(M == m, col_full, _BIG), axis=1,
                        keepdims=True)
    cand_part = jnp.min(jnp.where(z_p == m, l_io_p + _NT_FULL * 128, _BIG),
                        axis=1, keepdims=True)
    idx = jnp.minimum(cand_main, cand_part)

    l_io_w = jax.lax.broadcasted_iota(jnp.int32, (_RB, _CW), 1)

    def wstep(j, _):
        off = j * _CW
        out_ref[:, pl.ds(off, _CW)] = jnp.where(
            l_io_w + off == idx, 1.0, 0.0).astype(jnp.float32)
        return 0

    jax.lax.fori_loop(0, _NWCH, wstep, 0)
    l_io_t = jax.lax.broadcasted_iota(jnp.int32, (_RB, _WTAIL), 1)
    out_ref[:, pl.ds(_NWCH * _CW, _WTAIL)] = jnp.where(
        l_io_t + _NWCH * _CW == idx, 1.0, 0.0).astype(jnp.float32)


def kernel(logits):
    g = _gumbel_tiles()
    return pl.pallas_call(
        _body,
        grid=(_NBLK,),
        in_specs=[
            pl.BlockSpec((_RB, _COLS), lambda i: (i, 0)),
            pl.BlockSpec((1, _NT_PAD, _RB, 128), lambda i: (i, 0, 0, 0)),
        ],
        out_specs=pl.BlockSpec((_RB, _COLS), lambda i: (i, 0)),
        out_shape=jax.ShapeDtypeStruct((_ROWS, _COLS), jnp.float32),
    )(logits, g)


# speedup vs baseline: 1.1697x; 1.1697x over previous
import jax
import jax.numpy as jnp
import numpy as np
from jax.experimental import pallas as pl

_ROWS, _COLS = 128, 100000
_RB = 8
_CW = 2048
_NCH = _COLS // _CW
_TAIL = _COLS - _NCH * _CW
_BIG = np.int32(2 ** 30)

_KS1 = np.int32(42)
_KS2 = np.int32(0x1BD11BDA ^ 42)
_INJ = [
    (_KS1, np.int32(_KS2 + 1)),
    (_KS2, np.int32(2)),
    (np.int32(0), np.int32(_KS1 + 3)),
    (_KS1, np.int32(_KS2 + 4)),
    (_KS2, np.int32(5)),
]
_ROTS = [[13, 15, 26, 6], [17, 29, 16, 24]]


def _rotl(x, r):
    return jax.lax.bitwise_or(
        jax.lax.shift_left(x, np.int32(r)),
        jax.lax.shift_right_logical(x, np.int32(32 - r)),
    )


def _gumbel_from_counts(fl):
    x1 = fl + _KS1
    x0 = x1
    x1 = jax.lax.bitwise_xor(_rotl(x1, _ROTS[0][0]), x0)
    for r in _ROTS[0][1:]:
        x0 = x0 + x1
        x1 = jax.lax.bitwise_xor(_rotl(x1, r), x0)
    x0 = x0 + _INJ[0][0]
    x1 = x1 + _INJ[0][1]
    for grp in range(1, 5):
        for r in _ROTS[grp % 2]:
            x0 = x0 + x1
            x1 = jax.lax.bitwise_xor(_rotl(x1, r), x0)
        if int(_INJ[grp][0]) != 0:
            x0 = x0 + _INJ[grp][0]
        x1 = x1 + _INJ[grp][1]
    bits = jax.lax.bitwise_xor(x0, x1)
    ubits = jax.lax.bitwise_or(
        jax.lax.shift_right_logical(bits, np.int32(9)), np.int32(0x3F800000))
    u = jax.lax.bitcast_convert_type(ubits, jnp.float32) - 1.0
    inner = -jnp.log(u + 1e-10) + 1e-10
    return -jnp.log(inner)


def _body(x_ref, out_ref):
    pid = pl.program_id(0)
    row0 = pid * (_RB * _COLS)

    r_io = jax.lax.broadcasted_iota(jnp.int32, (_RB, _CW), 0)
    l_io = jax.lax.broadcasted_iota(jnp.int32, (_RB, _CW), 1)
    base_vec = r_io * _COLS + l_io

    def step(j, carry):
        m_acc, c_acc = carry
        off = j * _CW
        fl = base_vec + (row0 + off)
        g = _gumbel_from_counts(fl)
        z = x_ref[:, pl.ds(off, _CW)] + g
        upd = z > m_acc
        m_acc = jnp.where(upd, z, m_acc)
        c_acc = jnp.where(upd, l_io + off, c_acc)
        return m_acc, c_acc

    m0 = jnp.full((_RB, _CW), -jnp.inf, jnp.float32)
    c0 = jnp.zeros((_RB, _CW), jnp.int32)
    m_acc, c_acc = jax.lax.fori_loop(0, _NCH, step, (m0, c0))

    toff = _NCH * _CW
    r_io_t = jax.lax.broadcasted_iota(jnp.int32, (_RB, _TAIL), 0)
    l_io_t = jax.lax.broadcasted_iota(jnp.int32, (_RB, _TAIL), 1)
    fl_t = r_io_t * _COLS + l_io_t + (row0 + toff)
    g_t = _gumbel_from_counts(fl_t)
    z_t = x_ref[:, pl.ds(toff, _TAIL)] + g_t

    m_main = jnp.max(m_acc, axis=1, keepdims=True)
    m_tail = jnp.max(z_t, axis=1, keepdims=True)
    m = jnp.maximum(m_main, m_tail)
    cand_main = jnp.min(jnp.where(m_acc == m, c_acc, _BIG), axis=1,
                        keepdims=True)
    cand_tail = jnp.min(jnp.where(z_t == m, l_io_t + toff, _BIG), axis=1,
                        keepdims=True)
    idx = jnp.minimum(cand_main, cand_tail)

    def wstep(j, _):
        off = j * _CW
        out_ref[:, pl.ds(off, _CW)] = jnp.where(
            l_io + off == idx, 1.0, 0.0).astype(jnp.float32)
        return 0

    jax.lax.fori_loop(0, _NCH, wstep, 0)
    out_ref[:, pl.ds(toff, _TAIL)] = jnp.where(
        l_io_t + toff == idx, 1.0, 0.0).astype(jnp.float32)


def kernel(logits):
    return pl.pallas_call(
        _body,
        grid=(_ROWS // _RB,),
        in_specs=[pl.BlockSpec((_RB, _COLS), lambda i: (i, 0))],
        out_specs=pl.BlockSpec((_RB, _COLS), lambda i: (i, 0)),
        out_shape=jax.ShapeDtypeStruct((_ROWS, _COLS), jnp.float32),
    )(logits)
